# baseline (speedup 1.0000x reference)
import functools

import jax
import jax.numpy as jnp
from jax import lax
from jax.experimental import pallas as pl
from jax.experimental.pallas import tpu as pltpu
from jax.experimental.pallas import tpu_sc as plsc

N = 10000
E = 320000
NC = 2
NS = 16
NW = NC * NS
EPT = E // NW
C = 125
NCH = EPT // C
RPT = 640
NPAD = NS * RPT
NBUF = 4
BLK = 2000
GRID = N // BLK

_F32 = jnp.float32


@functools.lru_cache(maxsize=None)
def _make_deg():
    @functools.partial(
        pl.kernel,
        out_type=[jax.ShapeDtypeStruct((NPAD, 8), _F32)] * 2,
        mesh=plsc.VectorSubcoreMesh(core_axis_name="c", subcore_axis_name="s"),
        scratch_types=[
            pltpu.VMEM((NCH, C), jnp.int32),
            pltpu.VMEM((C, 8), _F32),
            pltpu.VMEM_SHARED((NPAD, 8), _F32),
        ],
        compiler_params=pltpu.CompilerParams(use_tc_tiling_on_sc=False),
    )
    def deg_kernel(dst_hbm, zeros_hbm, ones_hbm, out0_hbm, out1_hbm,
                   dst_v, buf, acc):
        core = lax.axis_index("c")
        sid = lax.axis_index("s")
        wid = sid * NC + core
        pltpu.sync_copy(dst_hbm.at[wid], dst_v)
        pltpu.sync_copy(zeros_hbm, acc.at[pl.ds(sid * RPT, RPT)])
        pltpu.sync_copy(ones_hbm, buf)
        plsc.subcore_barrier()

        def body(j, carry):
            pltpu.sync_copy(buf, acc.at[dst_v.at[j]], add=True)
            return carry

        lax.fori_loop(0, NCH, body, 0)
        plsc.subcore_barrier()

        @pl.when(core == 0)
        def _():
            pltpu.sync_copy(acc.at[pl.ds(sid * RPT, RPT)],
                            out0_hbm.at[pl.ds(sid * RPT, RPT)])

        @pl.when(core == 1)
        def _():
            pltpu.sync_copy(acc.at[pl.ds(sid * RPT, RPT)],
                            out1_hbm.at[pl.ds(sid * RPT, RPT)])

    return deg_kernel


@functools.lru_cache(maxsize=None)
def _make_prop2():
    @functools.partial(
        pl.kernel,
        out_type=[jax.ShapeDtypeStruct((NPAD, 64), _F32)] * 4,
        mesh=plsc.VectorSubcoreMesh(core_axis_name="c", subcore_axis_name="s"),
        scratch_types=(
            [pltpu.VMEM((NCH, C), jnp.int32)] * 2
            + [pltpu.VMEM((C, 64), _F32)] * NBUF
            + [pltpu.VMEM_SHARED((NPAD, 64), _F32)]
            + [pltpu.SemaphoreType.DMA] * NBUF
        ),
        compiler_params=pltpu.CompilerParams(use_tc_tiling_on_sc=False),
    )
    def prop2_kernel(hlo_hbm, hhi_hbm, src_hbm, dst_hbm, zeros_hbm,
                     olo0_hbm, olo1_hbm, ohi0_hbm, ohi1_hbm,
                     src_v, dst_v, *rest):
        bufs = rest[:NBUF]
        acc = rest[NBUF]
        sems = rest[NBUF + 1:]
        core = lax.axis_index("c")
        sid = lax.axis_index("s")
        wid = sid * NC + core
        pltpu.sync_copy(src_hbm.at[wid], src_v)
        pltpu.sync_copy(dst_hbm.at[wid], dst_v)

        for h_hbm, out0_hbm, out1_hbm in ((hlo_hbm, olo0_hbm, olo1_hbm),
                                          (hhi_hbm, ohi0_hbm, ohi1_hbm)):
            pltpu.sync_copy(zeros_hbm, acc.at[pl.ds(sid * RPT, RPT)])
            plsc.subcore_barrier()

            for i in range(NBUF):
                pltpu.async_copy(h_hbm.at[src_v.at[i]], bufs[i], sems[i])

            def body(t, carry):
                base = t * NBUF
                for i in range(NBUF):
                    ch = base + i
                    pltpu.make_async_copy(
                        h_hbm.at[src_v.at[0]], bufs[i], sems[i]).wait()
                    pltpu.sync_copy(bufs[i], acc.at[dst_v.at[ch]], add=True)
                    nxt = lax.rem(ch + NBUF, NCH)
                    pltpu.async_copy(h_hbm.at[src_v.at[nxt]], bufs[i],
                                     sems[i])
                return carry

            lax.fori_loop(0, NCH // NBUF, body, 0)
            for i in range(NBUF):
                pltpu.make_async_copy(
                    h_hbm.at[src_v.at[0]], bufs[i], sems[i]).wait()
            plsc.subcore_barrier()

            @pl.when(core == 0)
            def _():
                pltpu.sync_copy(acc.at[pl.ds(sid * RPT, RPT)],
                                out0_hbm.at[pl.ds(sid * RPT, RPT)])

            @pl.when(core == 1)
            def _():
                pltpu.sync_copy(acc.at[pl.ds(sid * RPT, RPT)],
                                out1_hbm.at[pl.ds(sid * RPT, RPT)])

    return prop2_kernel


@functools.lru_cache(maxsize=None)
def _make_prop(D):
    @functools.partial(
        pl.kernel,
        out_type=[jax.ShapeDtypeStruct((NPAD, D), _F32)] * 2,
        mesh=plsc.VectorSubcoreMesh(core_axis_name="c", subcore_axis_name="s"),
        scratch_types=(
            [pltpu.VMEM((NCH, C), jnp.int32)] * 2
            + [pltpu.VMEM((C, D), _F32)] * NBUF
            + [pltpu.VMEM_SHARED((NPAD, D), _F32)]
            + [pltpu.SemaphoreType.DMA] * NBUF
        ),
        compiler_params=pltpu.CompilerParams(use_tc_tiling_on_sc=False),
    )
    def prop_kernel(h_hbm, src_hbm, dst_hbm, zeros_hbm, out0_hbm, out1_hbm,
                    src_v, dst_v, *rest):
        bufs = rest[:NBUF]
        acc = rest[NBUF]
        sems = rest[NBUF + 1:]
        core = lax.axis_index("c")
        sid = lax.axis_index("s")
        wid = sid * NC + core
        pltpu.sync_copy(src_hbm.at[wid], src_v)
        pltpu.sync_copy(dst_hbm.at[wid], dst_v)
        pltpu.sync_copy(zeros_hbm, acc.at[pl.ds(sid * RPT, RPT)])
        plsc.subcore_barrier()

        for i in range(NBUF):
            pltpu.async_copy(h_hbm.at[src_v.at[i]], bufs[i], sems[i])

        def body(t, carry):
            base = t * NBUF
            for i in range(NBUF):
                ch = base + i
                pltpu.make_async_copy(
                    h_hbm.at[src_v.at[0]], bufs[i], sems[i]).wait()
                pltpu.sync_copy(bufs[i], acc.at[dst_v.at[ch]], add=True)
                nxt = lax.rem(ch + NBUF, NCH)
                pltpu.async_copy(h_hbm.at[src_v.at[nxt]], bufs[i], sems[i])
            return carry

        lax.fori_loop(0, NCH // NBUF, body, 0)
        for i in range(NBUF):
            pltpu.make_async_copy(
                h_hbm.at[src_v.at[0]], bufs[i], sems[i]).wait()
        plsc.subcore_barrier()

        @pl.when(core == 0)
        def _():
            pltpu.sync_copy(acc.at[pl.ds(sid * RPT, RPT)],
                            out0_hbm.at[pl.ds(sid * RPT, RPT)])

        @pl.when(core == 1)
        def _():
            pltpu.sync_copy(acc.at[pl.ds(sid * RPT, RPT)],
                            out1_hbm.at[pl.ds(sid * RPT, RPT)])

    return prop_kernel


def _rows(d):
    return pl.BlockSpec((BLK, d), lambda i: (i, 0))


def _whole(a, b):
    return pl.BlockSpec((a, b), lambda i: (0, 0))


_TC_PARAMS = pltpu.CompilerParams(dimension_semantics=("arbitrary",))


def _psum(a0_ref, a1_ref, h_ref, dinv):
    return (a0_ref[...] + a1_ref[...] + h_ref[...]) * dinv


def _tc_pre_body(deg0_ref, deg1_ref, x_ref, w1_ref, dinv_ref,
                 g1p_lo_ref, g1p_hi_ref):
    deg = deg0_ref[:, 0:1] + deg1_ref[:, 0:1] + 1.0
    dinv = jax.lax.rsqrt(deg)
    dinv_ref[...] = dinv
    g1 = jnp.dot(x_ref[...], w1_ref[...], preferred_element_type=_F32)
    g1p = g1 * dinv
    g1p_lo_ref[...] = g1p[:, 0:64]
    g1p_hi_ref[...] = g1p[:, 64:128]


def _tc_pre(deg0, deg1, x, W1):
    return pl.pallas_call(
        _tc_pre_body,
        grid=(GRID,),
        in_specs=[_rows(8), _rows(8), _rows(128), _whole(128, 128)],
        out_specs=[_rows(1), _rows(64), _rows(64)],
        out_shape=[
            jax.ShapeDtypeStruct((N, 1), _F32),
            jax.ShapeDtypeStruct((N, 64), _F32),
            jax.ShapeDtypeStruct((N, 64), _F32),
        ],
        compiler_params=_TC_PARAMS,
    )(deg0, deg1, x, W1)


def _kl_sum(xh, attf):
    s = jnp.sum(xh * attf, axis=1, keepdims=True) * (1.0 / 128.0)
    a = jnp.where(s >= 0, s, 0.2 * s)
    a = jnp.clip(jax.nn.sigmoid(a), 0.01, 0.99)
    return jnp.sum(a * jnp.log(2.0 * a) + (1.0 - a) * jnp.log(2.0 * (1.0 - a)))


def _tc_l1_body(lo0_ref, lo1_ref, hi0_ref, hi1_ref, g1p_lo_ref, g1p_hi_ref,
                dinv_ref, b1_ref, attf1_ref, w2p_ref, w3_ref,
                g2p_ref, g3p_lo_ref, g3p_hi_ref, kl1_ref):
    dinv = dinv_ref[...]
    p0_lo = _psum(lo0_ref, lo1_ref, g1p_lo_ref, dinv)
    p0_hi = _psum(hi0_ref, hi1_ref, g1p_hi_ref, dinv)
    p0 = jnp.concatenate([p0_lo, p0_hi], axis=1)
    x1 = jnp.maximum(p0 + b1_ref[...], 0.0)
    g2p_ref[...] = jnp.dot(
        x1, w2p_ref[...], preferred_element_type=_F32) * dinv
    g3p = jnp.dot(x1, w3_ref[...], preferred_element_type=_F32) * dinv
    g3p_lo_ref[...] = g3p[:, 0:64]
    g3p_hi_ref[...] = g3p[:, 64:128]

    @pl.when(pl.program_id(0) == 0)
    def _():
        kl1_ref[...] = jnp.zeros_like(kl1_ref)

    kl1_ref[...] += jnp.reshape(_kl_sum(x1, attf1_ref[...]), (1, 1))


def _tc_l1(acc0_lo, acc1_lo, acc0_hi, acc1_hi, g1p_lo, g1p_hi, dinv, b1r,
           attf1, W2p, W3):
    return pl.pallas_call(
        _tc_l1_body,
        grid=(GRID,),
        in_specs=[_rows(64)] * 6 + [_rows(1), _whole(1, 128), _rows(128),
                                    _whole(128, 8), _whole(128, 128)],
        out_specs=[_rows(8), _rows(64), _rows(64), _whole(1, 1)],
        out_shape=[
            jax.ShapeDtypeStruct((N, 8), _F32),
            jax.ShapeDtypeStruct((N, 64), _F32),
            jax.ShapeDtypeStruct((N, 64), _F32),
            jax.ShapeDtypeStruct((1, 1), _F32),
        ],
        compiler_params=_TC_PARAMS,
    )(acc0_lo, acc1_lo, acc0_hi, acc1_hi, g1p_lo, g1p_hi, dinv, b1r,
      attf1, W2p, W3)


def _tc_l2_body(a2_0_ref, a2_1_ref, lo0_ref, lo1_ref, hi0_ref, hi1_ref,
                g2p_ref, g3p_lo_ref, g3p_hi_ref, dinv_ref, b2p_ref,
                b3_ref, attf2_ref, w4p_ref, y1p_ref, g4p_ref, kl2_ref):
    dinv = dinv_ref[...]
    y1p_ref[...] = _psum(a2_0_ref, a2_1_ref, g2p_ref, dinv) + b2p_ref[...]
    p3_lo = _psum(lo0_ref, lo1_ref, g3p_lo_ref, dinv)
    p3_hi = _psum(hi0_ref, hi1_ref, g3p_hi_ref, dinv)
    x2 = jnp.maximum(
        jnp.concatenate([p3_lo, p3_hi], axis=1) + b3_ref[...], 0.0)
    g4p_ref[...] = jnp.dot(
        x2, w4p_ref[...], preferred_element_type=_F32) * dinv

    @pl.when(pl.program_id(0) == 0)
    def _():
        kl2_ref[...] = jnp.zeros_like(kl2_ref)

    kl2_ref[...] += jnp.reshape(_kl_sum(x2, attf2_ref[...]), (1, 1))


def _tc_l2(a2_0, a2_1, acc0_lo, acc1_lo, acc0_hi, acc1_hi, g2p, g3p_lo,
           g3p_hi, dinv, b2pr, b3r, attf2, W4p):
    return pl.pallas_call(
        _tc_l2_body,
        grid=(GRID,),
        in_specs=([_rows(8), _rows(8)] + [_rows(64)] * 4
                  + [_rows(8), _rows(64), _rows(64), _rows(1),
                     _whole(1, 8), _whole(1, 128), _rows(128),
                     _whole(128, 8)]),
        out_specs=[_rows(8), _rows(8), _whole(1, 1)],
        out_shape=[
            jax.ShapeDtypeStruct((N, 8), _F32),
            jax.ShapeDtypeStruct((N, 8), _F32),
            jax.ShapeDtypeStruct((1, 1), _F32),
        ],
        compiler_params=_TC_PARAMS,
    )(a2_0, a2_1, acc0_lo, acc1_lo, acc0_hi, acc1_hi, g2p, g3p_lo, g3p_hi,
      dinv, b2pr, b3r, attf2, W4p)


def _tc_fin_body(a0_ref, a1_ref, h4p_ref, dinv_ref, b4p_ref, kl1_ref,
                 kl2_ref, y2p_ref, kl_ref):
    y2 = _psum(a0_ref, a1_ref, h4p_ref, dinv_ref[...])
    y2p_ref[...] = y2 + b4p_ref[...]
    kl_ref[...] = (kl1_ref[...] + kl2_ref[...]) * 0.5


def _tc_fin(acc0, acc1, h4p, dinv, b4pr, kl1, kl2):
    return pl.pallas_call(
        _tc_fin_body,
        grid=(GRID,),
        in_specs=[_rows(8), _rows(8), _rows(8), _rows(1), _whole(1, 8),
                  _whole(1, 1), _whole(1, 1)],
        out_specs=[_rows(8), _whole(1, 1)],
        out_shape=[
            jax.ShapeDtypeStruct((N, 8), _F32),
            jax.ShapeDtypeStruct((1, 1), _F32),
        ],
        compiler_params=_TC_PARAMS,
    )(acc0, acc1, h4p, dinv, b4pr, kl1, kl2)


def kernel(x, G, W1, b1, att1, W2, b2, W3, b3, att2, W4, b4):
    srcr = G[0].reshape(NW, NCH, C)
    dstr = G[1].reshape(NW, NCH, C)
    zeros64 = jnp.zeros((RPT, 64), _F32)
    zeros8 = jnp.zeros((RPT, 8), _F32)
    ones8 = jnp.ones((C, 8), _F32)
    attf1 = jnp.tile(att1, (N // 8, 1))
    attf2 = jnp.tile(att2, (N // 8, 1))
    W2p = jnp.pad(W2, ((0, 0), (0, 5)))
    b2pr = jnp.pad(b2, (0, 5)).reshape(1, 8)
    W4p = jnp.pad(W4, ((0, 0), (0, 5)))
    b4pr = jnp.pad(b4, (0, 5)).reshape(1, 8)
    b1r = b1.reshape(1, 128)
    b3r = b3.reshape(1, 128)

    deg0, deg1 = _make_deg()(dstr, zeros8, ones8)
    dinv, g1p_lo, g1p_hi = _tc_pre(deg0, deg1, x, W1)
    prop2 = _make_prop2()
    prop8 = _make_prop(8)
    a1lo_0, a1lo_1, a1hi_0, a1hi_1 = prop2(g1p_lo, g1p_hi, srcr, dstr,
                                           zeros64)
    g2p, g3p_lo, g3p_hi, kl1 = _tc_l1(a1lo_0, a1lo_1, a1hi_0, a1hi_1,
                                      g1p_lo, g1p_hi, dinv, b1r, attf1,
                                      W2p, W3)
    a2_0, a2_1 = prop8(g2p, srcr, dstr, zeros8)
    a3lo_0, a3lo_1, a3hi_0, a3hi_1 = prop2(g3p_lo, g3p_hi, srcr, dstr,
                                           zeros64)
    y1p, g4p, kl2 = _tc_l2(a2_0, a2_1, a3lo_0, a3lo_1, a3hi_0, a3hi_1,
                           g2p, g3p_lo, g3p_hi, dinv, b2pr, b3r, attf2,
                           W4p)
    a4_0, a4_1 = prop8(g4p, srcr, dstr, zeros8)
    y2p, kl = _tc_fin(a4_0, a4_1, g4p, dinv, b4pr, kl1, kl2)
    return y1p[:, :3], y2p[:, :3], kl[0, 0]


# speedup vs baseline: 1.0251x; 1.0251x over previous
import functools

import jax
import jax.numpy as jnp
from jax import lax
from jax.experimental import pallas as pl
from jax.experimental.pallas import tpu as pltpu
from jax.experimental.pallas import tpu_sc as plsc

N = 10000
E = 320000
NC = 2
NS = 16
NW = NC * NS
EPT = E // NW
C = 125
NCH = EPT // C
RPT = 640
NPAD = NS * RPT
NBUF = 8
BLK = 2000
GRID = N // BLK

_F32 = jnp.float32


@functools.lru_cache(maxsize=None)
def _make_deg():
    @functools.partial(
        pl.kernel,
        out_type=[jax.ShapeDtypeStruct((NPAD, 8), _F32)] * 2,
        mesh=plsc.VectorSubcoreMesh(core_axis_name="c", subcore_axis_name="s"),
        scratch_types=[
            pltpu.VMEM((NCH, C), jnp.int32),
            pltpu.VMEM((C, 8), _F32),
            pltpu.VMEM_SHARED((NPAD, 8), _F32),
        ],
        compiler_params=pltpu.CompilerParams(use_tc_tiling_on_sc=False),
    )
    def deg_kernel(dst_hbm, zeros_hbm, ones_hbm, out0_hbm, out1_hbm,
                   dst_v, buf, acc):
        core = lax.axis_index("c")
        sid = lax.axis_index("s")
        wid = sid * NC + core
        pltpu.sync_copy(dst_hbm.at[wid], dst_v)
        pltpu.sync_copy(zeros_hbm, acc.at[pl.ds(sid * RPT, RPT)])
        pltpu.sync_copy(ones_hbm, buf)
        plsc.subcore_barrier()

        def body(j, carry):
            pltpu.sync_copy(buf, acc.at[dst_v.at[j]], add=True)
            return carry

        lax.fori_loop(0, NCH, body, 0)
        plsc.subcore_barrier()

        @pl.when(core == 0)
        def _():
            pltpu.sync_copy(acc.at[pl.ds(sid * RPT, RPT)],
                            out0_hbm.at[pl.ds(sid * RPT, RPT)])

        @pl.when(core == 1)
        def _():
            pltpu.sync_copy(acc.at[pl.ds(sid * RPT, RPT)],
                            out1_hbm.at[pl.ds(sid * RPT, RPT)])

    return deg_kernel


@functools.lru_cache(maxsize=None)
def _make_prop2():
    @functools.partial(
        pl.kernel,
        out_type=[jax.ShapeDtypeStruct((NPAD, 64), _F32)] * 4,
        mesh=plsc.VectorSubcoreMesh(core_axis_name="c", subcore_axis_name="s"),
        scratch_types=(
            [pltpu.VMEM((NCH, C), jnp.int32)] * 2
            + [pltpu.VMEM((C, 64), _F32)] * NBUF
            + [pltpu.VMEM_SHARED((NPAD, 64), _F32)]
            + [pltpu.SemaphoreType.DMA] * NBUF
        ),
        compiler_params=pltpu.CompilerParams(use_tc_tiling_on_sc=False),
    )
    def prop2_kernel(hlo_hbm, hhi_hbm, src_hbm, dst_hbm, zeros_hbm,
                     olo0_hbm, olo1_hbm, ohi0_hbm, ohi1_hbm,
                     src_v, dst_v, *rest):
        bufs = rest[:NBUF]
        acc = rest[NBUF]
        sems = rest[NBUF + 1:]
        core = lax.axis_index("c")
        sid = lax.axis_index("s")
        wid = sid * NC + core
        pltpu.sync_copy(src_hbm.at[wid], src_v)
        pltpu.sync_copy(dst_hbm.at[wid], dst_v)

        for h_hbm, out0_hbm, out1_hbm in ((hlo_hbm, olo0_hbm, olo1_hbm),
                                          (hhi_hbm, ohi0_hbm, ohi1_hbm)):
            pltpu.sync_copy(zeros_hbm, acc.at[pl.ds(sid * RPT, RPT)])
            plsc.subcore_barrier()

            for i in range(NBUF):
                pltpu.async_copy(h_hbm.at[src_v.at[i]], bufs[i], sems[i])

            def body(t, carry):
                base = t * NBUF
                for i in range(NBUF):
                    ch = base + i
                    pltpu.make_async_copy(
                        h_hbm.at[src_v.at[0]], bufs[i], sems[i]).wait()
                    pltpu.sync_copy(bufs[i], acc.at[dst_v.at[ch]], add=True)
                    nxt = lax.rem(ch + NBUF, NCH)
                    pltpu.async_copy(h_hbm.at[src_v.at[nxt]], bufs[i],
                                     sems[i])
                return carry

            lax.fori_loop(0, NCH // NBUF, body, 0)
            for i in range(NBUF):
                pltpu.make_async_copy(
                    h_hbm.at[src_v.at[0]], bufs[i], sems[i]).wait()
            plsc.subcore_barrier()

            @pl.when(core == 0)
            def _():
                pltpu.sync_copy(acc.at[pl.ds(sid * RPT, RPT)],
                                out0_hbm.at[pl.ds(sid * RPT, RPT)])

            @pl.when(core == 1)
            def _():
                pltpu.sync_copy(acc.at[pl.ds(sid * RPT, RPT)],
                                out1_hbm.at[pl.ds(sid * RPT, RPT)])

    return prop2_kernel


@functools.lru_cache(maxsize=None)
def _make_prop(D):
    @functools.partial(
        pl.kernel,
        out_type=[jax.ShapeDtypeStruct((NPAD, D), _F32)] * 2,
        mesh=plsc.VectorSubcoreMesh(core_axis_name="c", subcore_axis_name="s"),
        scratch_types=(
            [pltpu.VMEM((NCH, C), jnp.int32)] * 2
            + [pltpu.VMEM((C, D), _F32)] * NBUF
            + [pltpu.VMEM_SHARED((NPAD, D), _F32)]
            + [pltpu.SemaphoreType.DMA] * NBUF
        ),
        compiler_params=pltpu.CompilerParams(use_tc_tiling_on_sc=False),
    )
    def prop_kernel(h_hbm, src_hbm, dst_hbm, zeros_hbm, out0_hbm, out1_hbm,
                    src_v, dst_v, *rest):
        bufs = rest[:NBUF]
        acc = rest[NBUF]
        sems = rest[NBUF + 1:]
        core = lax.axis_index("c")
        sid = lax.axis_index("s")
        wid = sid * NC + core
        pltpu.sync_copy(src_hbm.at[wid], src_v)
        pltpu.sync_copy(dst_hbm.at[wid], dst_v)
        pltpu.sync_copy(zeros_hbm, acc.at[pl.ds(sid * RPT, RPT)])
        plsc.subcore_barrier()

        for i in range(NBUF):
            pltpu.async_copy(h_hbm.at[src_v.at[i]], bufs[i], sems[i])

        def body(t, carry):
            base = t * NBUF
            for i in range(NBUF):
                ch = base + i
                pltpu.make_async_copy(
                    h_hbm.at[src_v.at[0]], bufs[i], sems[i]).wait()
                pltpu.sync_copy(bufs[i], acc.at[dst_v.at[ch]], add=True)
                nxt = lax.rem(ch + NBUF, NCH)
                pltpu.async_copy(h_hbm.at[src_v.at[nxt]], bufs[i], sems[i])
            return carry

        lax.fori_loop(0, NCH // NBUF, body, 0)
        for i in range(NBUF):
            pltpu.make_async_copy(
                h_hbm.at[src_v.at[0]], bufs[i], sems[i]).wait()
        plsc.subcore_barrier()

        @pl.when(core == 0)
        def _():
            pltpu.sync_copy(acc.at[pl.ds(sid * RPT, RPT)],
                            out0_hbm.at[pl.ds(sid * RPT, RPT)])

        @pl.when(core == 1)
        def _():
            pltpu.sync_copy(acc.at[pl.ds(sid * RPT, RPT)],
                            out1_hbm.at[pl.ds(sid * RPT, RPT)])

    return prop_kernel


def _rows(d):
    return pl.BlockSpec((BLK, d), lambda i: (i, 0))


def _whole(a, b):
    return pl.BlockSpec((a, b), lambda i: (0, 0))


_TC_PARAMS = pltpu.CompilerParams(dimension_semantics=("arbitrary",))


def _psum(a0_ref, a1_ref, h_ref, dinv):
    return (a0_ref[...] + a1_ref[...] + h_ref[...]) * dinv


def _tc_pre_body(deg0_ref, deg1_ref, x_ref, w1_ref, dinv_ref,
                 g1p_lo_ref, g1p_hi_ref):
    deg = deg0_ref[:, 0:1] + deg1_ref[:, 0:1] + 1.0
    dinv = jax.lax.rsqrt(deg)
    dinv_ref[...] = dinv
    g1 = jnp.dot(x_ref[...], w1_ref[...], preferred_element_type=_F32)
    g1p = g1 * dinv
    g1p_lo_ref[...] = g1p[:, 0:64]
    g1p_hi_ref[...] = g1p[:, 64:128]


def _tc_pre(deg0, deg1, x, W1):
    return pl.pallas_call(
        _tc_pre_body,
        grid=(GRID,),
        in_specs=[_rows(8), _rows(8), _rows(128), _whole(128, 128)],
        out_specs=[_rows(1), _rows(64), _rows(64)],
        out_shape=[
            jax.ShapeDtypeStruct((N, 1), _F32),
            jax.ShapeDtypeStruct((N, 64), _F32),
            jax.ShapeDtypeStruct((N, 64), _F32),
        ],
        compiler_params=_TC_PARAMS,
    )(deg0, deg1, x, W1)


def _kl_sum(xh, attf):
    s = jnp.sum(xh * attf, axis=1, keepdims=True) * (1.0 / 128.0)
    a = jnp.where(s >= 0, s, 0.2 * s)
    a = jnp.clip(jax.nn.sigmoid(a), 0.01, 0.99)
    return jnp.sum(a * jnp.log(2.0 * a) + (1.0 - a) * jnp.log(2.0 * (1.0 - a)))


def _tc_l1_body(lo0_ref, lo1_ref, hi0_ref, hi1_ref, g1p_lo_ref, g1p_hi_ref,
                dinv_ref, b1_ref, attf1_ref, w2p_ref, w3_ref,
                g2p_ref, g3p_lo_ref, g3p_hi_ref, kl1_ref):
    dinv = dinv_ref[...]
    p0_lo = _psum(lo0_ref, lo1_ref, g1p_lo_ref, dinv)
    p0_hi = _psum(hi0_ref, hi1_ref, g1p_hi_ref, dinv)
    p0 = jnp.concatenate([p0_lo, p0_hi], axis=1)
    x1 = jnp.maximum(p0 + b1_ref[...], 0.0)
    g2p_ref[...] = jnp.dot(
        x1, w2p_ref[...], preferred_element_type=_F32) * dinv
    g3p = jnp.dot(x1, w3_ref[...], preferred_element_type=_F32) * dinv
    g3p_lo_ref[...] = g3p[:, 0:64]
    g3p_hi_ref[...] = g3p[:, 64:128]

    @pl.when(pl.program_id(0) == 0)
    def _():
        kl1_ref[...] = jnp.zeros_like(kl1_ref)

    kl1_ref[...] += jnp.reshape(_kl_sum(x1, attf1_ref[...]), (1, 1))


def _tc_l1(acc0_lo, acc1_lo, acc0_hi, acc1_hi, g1p_lo, g1p_hi, dinv, b1r,
           attf1, W2p, W3):
    return pl.pallas_call(
        _tc_l1_body,
        grid=(GRID,),
        in_specs=[_rows(64)] * 6 + [_rows(1), _whole(1, 128), _rows(128),
                                    _whole(128, 8), _whole(128, 128)],
        out_specs=[_rows(8), _rows(64), _rows(64), _whole(1, 1)],
        out_shape=[
            jax.ShapeDtypeStruct((N, 8), _F32),
            jax.ShapeDtypeStruct((N, 64), _F32),
            jax.ShapeDtypeStruct((N, 64), _F32),
            jax.ShapeDtypeStruct((1, 1), _F32),
        ],
        compiler_params=_TC_PARAMS,
    )(acc0_lo, acc1_lo, acc0_hi, acc1_hi, g1p_lo, g1p_hi, dinv, b1r,
      attf1, W2p, W3)


def _tc_l2_body(a2_0_ref, a2_1_ref, lo0_ref, lo1_ref, hi0_ref, hi1_ref,
                g2p_ref, g3p_lo_ref, g3p_hi_ref, dinv_ref, b2p_ref,
                b3_ref, attf2_ref, w4p_ref, y1p_ref, g4p_ref, kl2_ref):
    dinv = dinv_ref[...]
    y1p_ref[...] = _psum(a2_0_ref, a2_1_ref, g2p_ref, dinv) + b2p_ref[...]
    p3_lo = _psum(lo0_ref, lo1_ref, g3p_lo_ref, dinv)
    p3_hi = _psum(hi0_ref, hi1_ref, g3p_hi_ref, dinv)
    x2 = jnp.maximum(
        jnp.concatenate([p3_lo, p3_hi], axis=1) + b3_ref[...], 0.0)
    g4p_ref[...] = jnp.dot(
        x2, w4p_ref[...], preferred_element_type=_F32) * dinv

    @pl.when(pl.program_id(0) == 0)
    def _():
        kl2_ref[...] = jnp.zeros_like(kl2_ref)

    kl2_ref[...] += jnp.reshape(_kl_sum(x2, attf2_ref[...]), (1, 1))


def _tc_l2(a2_0, a2_1, acc0_lo, acc1_lo, acc0_hi, acc1_hi, g2p, g3p_lo,
           g3p_hi, dinv, b2pr, b3r, attf2, W4p):
    return pl.pallas_call(
        _tc_l2_body,
        grid=(GRID,),
        in_specs=([_rows(8), _rows(8)] + [_rows(64)] * 4
                  + [_rows(8), _rows(64), _rows(64), _rows(1),
                     _whole(1, 8), _whole(1, 128), _rows(128),
                     _whole(128, 8)]),
        out_specs=[_rows(8), _rows(8), _whole(1, 1)],
        out_shape=[
            jax.ShapeDtypeStruct((N, 8), _F32),
            jax.ShapeDtypeStruct((N, 8), _F32),
            jax.ShapeDtypeStruct((1, 1), _F32),
        ],
        compiler_params=_TC_PARAMS,
    )(a2_0, a2_1, acc0_lo, acc1_lo, acc0_hi, acc1_hi, g2p, g3p_lo, g3p_hi,
      dinv, b2pr, b3r, attf2, W4p)


def _tc_fin_body(a0_ref, a1_ref, h4p_ref, dinv_ref, b4p_ref, kl1_ref,
                 kl2_ref, y2p_ref, kl_ref):
    y2 = _psum(a0_ref, a1_ref, h4p_ref, dinv_ref[...])
    y2p_ref[...] = y2 + b4p_ref[...]
    kl_ref[...] = (kl1_ref[...] + kl2_ref[...]) * 0.5


def _tc_fin(acc0, acc1, h4p, dinv, b4pr, kl1, kl2):
    return pl.pallas_call(
        _tc_fin_body,
        grid=(GRID,),
        in_specs=[_rows(8), _rows(8), _rows(8), _rows(1), _whole(1, 8),
                  _whole(1, 1), _whole(1, 1)],
        out_specs=[_rows(8), _whole(1, 1)],
        out_shape=[
            jax.ShapeDtypeStruct((N, 8), _F32),
            jax.ShapeDtypeStruct((1, 1), _F32),
        ],
        compiler_params=_TC_PARAMS,
    )(acc0, acc1, h4p, dinv, b4pr, kl1, kl2)


def kernel(x, G, W1, b1, att1, W2, b2, W3, b3, att2, W4, b4):
    srcr = G[0].reshape(NW, NCH, C)
    dstr = G[1].reshape(NW, NCH, C)
    zeros64 = jnp.zeros((RPT, 64), _F32)
    zeros8 = jnp.zeros((RPT, 8), _F32)
    ones8 = jnp.ones((C, 8), _F32)
    attf1 = jnp.tile(att1, (N // 8, 1))
    attf2 = jnp.tile(att2, (N // 8, 1))
    W2p = jnp.pad(W2, ((0, 0), (0, 5)))
    b2pr = jnp.pad(b2, (0, 5)).reshape(1, 8)
    W4p = jnp.pad(W4, ((0, 0), (0, 5)))
    b4pr = jnp.pad(b4, (0, 5)).reshape(1, 8)
    b1r = b1.reshape(1, 128)
    b3r = b3.reshape(1, 128)

    deg0, deg1 = _make_deg()(dstr, zeros8, ones8)
    dinv, g1p_lo, g1p_hi = _tc_pre(deg0, deg1, x, W1)
    prop2 = _make_prop2()
    prop8 = _make_prop(8)
    a1lo_0, a1lo_1, a1hi_0, a1hi_1 = prop2(g1p_lo, g1p_hi, srcr, dstr,
                                           zeros64)
    g2p, g3p_lo, g3p_hi, kl1 = _tc_l1(a1lo_0, a1lo_1, a1hi_0, a1hi_1,
                                      g1p_lo, g1p_hi, dinv, b1r, attf1,
                                      W2p, W3)
    a2_0, a2_1 = prop8(g2p, srcr, dstr, zeros8)
    a3lo_0, a3lo_1, a3hi_0, a3hi_1 = prop2(g3p_lo, g3p_hi, srcr, dstr,
                                           zeros64)
    y1p, g4p, kl2 = _tc_l2(a2_0, a2_1, a3lo_0, a3lo_1, a3hi_0, a3hi_1,
                           g2p, g3p_lo, g3p_hi, dinv, b2pr, b3r, attf2,
                           W4p)
    a4_0, a4_1 = prop8(g4p, srcr, dstr, zeros8)
    y2p, kl = _tc_fin(a4_0, a4_1, g4p, dinv, b4pr, kl1, kl2)
    return y1p[:, :3], y2p[:, :3], kl[0, 0]
